# baseline (speedup 1.0000x reference)
import jax
import jax.numpy as jnp
from jax import lax
from jax.experimental import pallas as pl
from jax.experimental.pallas import tpu as pltpu
from jax.experimental.pallas import tpu_sc as plsc

_B = 4096
_D = 32
_NCARD = 200
_NCR = 16
_NACT = 50
_ADEPTH = 4
_R = _B * _NCARD

_NW = 32
_CH = 512
_NSUB = _CH // 128
_UNITS = _R // _CH // _NW
_IROWS = _UNITS * _NSUB


def _sc_body(cards_ref, acts_ref, table_ref,
             card_out_ref, act_out_ref,
             idx_all, g0, g1,
             gsem0, gsem1, osem0, osem1):
  wid = lax.axis_index("s") * 2 + lax.axis_index("c")
  ubase = wid * _UNITS
  irow0 = wid * _IROWS

  def fire_gathers(u, rows_v, gsem):
    for j in range(_NSUB):
      pltpu.async_copy(table_ref.at[idx_all.at[u * _NSUB + j]],
                       rows_v.at[pl.ds(j * 128, 128)], gsem)

  def drain_gathers(rows_v, gsem):
    pltpu.make_async_copy(table_ref.at[pl.ds(0, _CH)], rows_v, gsem).wait()

  def card_out(u, rows_v, osem):
    g = ubase + u
    c = g >> 3
    b0 = (g & 7) * _CH
    return pltpu.async_copy(rows_v, card_out_ref.at[c, pl.ds(b0, _CH)], osem)

  def act_out(u, rows_v, osem):
    g = ubase + u
    a = g >> 5
    d = (g >> 3) & 3
    b0 = (g & 7) * _CH
    return pltpu.async_copy(
        rows_v, act_out_ref.at[a, pl.ds(b0, _CH), pl.ds(d * _D, _D)], osem)

  def phase(idx_hbm, out_fn):
    pltpu.sync_copy(idx_hbm.at[pl.ds(irow0, _IROWS)], idx_all)
    fire_gathers(0, g0, gsem0)
    fire_gathers(1, g1, gsem1)

    def pair(i, carry):
      u = 2 * i
      drain_gathers(g0, gsem0)
      out_fn(u, g0, osem0).wait()
      fire_gathers(u + 2, g0, gsem0)
      drain_gathers(g1, gsem1)
      out_fn(u + 1, g1, osem1).wait()
      fire_gathers(u + 3, g1, gsem1)
      return carry

    lax.fori_loop(0, _UNITS // 2 - 1, pair, 0)
    drain_gathers(g0, gsem0)
    out_fn(_UNITS - 2, g0, osem0).wait()
    drain_gathers(g1, gsem1)
    out_fn(_UNITS - 1, g1, osem1).wait()

  phase(cards_ref, card_out)
  phase(acts_ref, act_out)


def _sc_call(cards2, acts2, table):
  mesh = plsc.VectorSubcoreMesh(core_axis_name="c", subcore_axis_name="s",
                                num_cores=2, num_subcores=16)
  f = pl.kernel(
      _sc_body,
      out_type=(jax.ShapeDtypeStruct((_NCARD, _B, _D), jnp.float32),
                jax.ShapeDtypeStruct((_NACT, _B, _ADEPTH * _D), jnp.float32)),
      mesh=mesh,
      compiler_params=pltpu.CompilerParams(use_tc_tiling_on_sc=False,
                                           needs_layout_passes=False),
      scratch_types=(
          pltpu.VMEM((_IROWS, 128), jnp.int32),
          pltpu.VMEM((_CH, _D), jnp.float32),
          pltpu.VMEM((_CH, _D), jnp.float32),
          pltpu.SemaphoreType.DMA,
          pltpu.SemaphoreType.DMA,
          pltpu.SemaphoreType.DMA,
          pltpu.SemaphoreType.DMA,
      ),
  )
  return f(cards2, acts2, table)


def _card_tc_body(e_ref, n_ref, s_ref, b_ref, o_ref):
  emb = e_ref[0]
  ident = jnp.eye(_D, dtype=jnp.float32)
  o_ref[0, pl.ds(0, _D), :] = lax.dot_general(
      ident, emb, (((1,), (1,)), ((), ())),
      preferred_element_type=jnp.float32)
  o_ref[0, pl.ds(_D, _NCR), :] = n_ref[0] * s_ref[...] + b_ref[...]


def _card_tc(card_embed_t, nums_t, scale, bias):
  return pl.pallas_call(
      _card_tc_body,
      grid=(_NCARD,),
      in_specs=[
          pl.BlockSpec((1, _B, _D), lambda c: (c, 0, 0)),
          pl.BlockSpec((1, _NCR, _B), lambda c: (c, 0, 0)),
          pl.BlockSpec((_NCR, 1), lambda c: (0, 0)),
          pl.BlockSpec((_NCR, 1), lambda c: (0, 0)),
      ],
      out_specs=pl.BlockSpec((1, _D + _NCR, _B), lambda c: (c, 0, 0)),
      out_shape=jax.ShapeDtypeStruct((_NCARD, _D + _NCR, _B), jnp.float32),
  )(card_embed_t, nums_t, scale, bias)


def _reals_body(r_ref, a_ref, v_ref, o_ref):
  o_ref[...] = (r_ref[...] - a_ref[...]) / jnp.sqrt(v_ref[...])


def _reals_norm(reals, avg, var):
  return pl.pallas_call(
      _reals_body,
      out_shape=jax.ShapeDtypeStruct(reals.shape, reals.dtype),
  )(reals, avg, var)


def kernel(reals, cardIDs, card_nums, actionIDs, action_mask,
           embed_table, avg_reals, var_reals, avg_cards, var_cards):
  cards2 = cardIDs.astype(jnp.int32).T.reshape(_R // 128, 128)
  acts2 = actionIDs.astype(jnp.int32).transpose(1, 2, 0).reshape(_R // 128, 128)
  nums_t = card_nums.transpose(1, 2, 0)
  scale = (1.0 / jnp.sqrt(var_cards)).reshape(_NCR, 1)
  bias = (-avg_cards).reshape(_NCR, 1) * scale
  table_lin = lax.optimization_barrier(
      embed_table.reshape(250000, 128)).reshape(1000000, _D)
  card_embed_t, act_out = _sc_call(cards2, acts2, table_lin)
  card_out = _card_tc(card_embed_t, nums_t, scale, bias)
  reals_n = _reals_norm(reals, avg_reals, var_reals)
  card_all = card_out.transpose(2, 0, 1)
  action_embed = act_out.transpose(1, 0, 2)
  return (reals_n, card_all, action_embed, action_mask)


# speedup vs baseline: 1.0204x; 1.0204x over previous
import jax
import jax.numpy as jnp
from jax import lax
from jax.experimental import pallas as pl
from jax.experimental.pallas import tpu as pltpu
from jax.experimental.pallas import tpu_sc as plsc

_B = 4096
_D = 32
_NCARD = 200
_NCR = 16
_NACT = 50
_ADEPTH = 4
_R = _B * _NCARD

_NW = 32
_CH = 512
_NSUB = _CH // 128
_UNITS = _R // _CH // _NW
_IROWS = _UNITS * _NSUB


def _gather_body(kind):
  def body(idx_hbm, table_ref, out_ref,
           idx_all, g0, g1, gsem0, gsem1, osem0, osem1):
    wid = lax.axis_index("s") * 2 + lax.axis_index("c")
    ubase = wid * _UNITS
    irow0 = wid * _IROWS

    def fire_gathers(u, rows_v, gsem):
      for j in range(_NSUB):
        pltpu.async_copy(table_ref.at[idx_all.at[u * _NSUB + j]],
                         rows_v.at[pl.ds(j * 128, 128)], gsem)

    def drain_gathers(rows_v, gsem):
      pltpu.make_async_copy(table_ref.at[pl.ds(0, _CH)], rows_v, gsem).wait()

    def out_fn(u, rows_v, osem):
      g = ubase + u
      b0 = (g & 7) * _CH
      if kind == "card":
        return pltpu.async_copy(rows_v, out_ref.at[g >> 3, pl.ds(b0, _CH)],
                                osem)
      a = g >> 5
      d = (g >> 3) & 3
      return pltpu.async_copy(
          rows_v, out_ref.at[a, pl.ds(b0, _CH), pl.ds(d * _D, _D)], osem)

    pltpu.sync_copy(idx_hbm.at[pl.ds(irow0, _IROWS)], idx_all)
    fire_gathers(0, g0, gsem0)
    fire_gathers(1, g1, gsem1)

    def pair(i, carry):
      u = 2 * i
      drain_gathers(g0, gsem0)
      out_fn(u, g0, osem0).wait()
      fire_gathers(u + 2, g0, gsem0)
      drain_gathers(g1, gsem1)
      out_fn(u + 1, g1, osem1).wait()
      fire_gathers(u + 3, g1, gsem1)
      return carry

    lax.fori_loop(0, _UNITS // 2 - 1, pair, 0)
    drain_gathers(g0, gsem0)
    out_fn(_UNITS - 2, g0, osem0).wait()
    drain_gathers(g1, gsem1)
    out_fn(_UNITS - 1, g1, osem1).wait()

  return body


def _sc_gather(kind, out_shape, idx2, table):
  mesh = plsc.VectorSubcoreMesh(core_axis_name="c", subcore_axis_name="s",
                                num_cores=2, num_subcores=16)
  f = pl.kernel(
      _gather_body(kind),
      out_type=jax.ShapeDtypeStruct(out_shape, jnp.float32),
      mesh=mesh,
      compiler_params=pltpu.CompilerParams(use_tc_tiling_on_sc=False,
                                           needs_layout_passes=False),
      scratch_types=(
          pltpu.VMEM((_IROWS, 128), jnp.int32),
          pltpu.VMEM((_CH, _D), jnp.float32),
          pltpu.VMEM((_CH, _D), jnp.float32),
          pltpu.SemaphoreType.DMA,
          pltpu.SemaphoreType.DMA,
          pltpu.SemaphoreType.DMA,
          pltpu.SemaphoreType.DMA,
      ),
  )
  return f(idx2, table)


def _sc_call(cards2, acts2, table):
  card = _sc_gather("card", (_NCARD, _B, _D), cards2, table)
  act = _sc_gather("act", (_NACT, _B, _ADEPTH * _D), acts2, table)
  return card, act


def _card_tc_body(e_ref, n_ref, s_ref, b_ref, o_ref):
  emb = e_ref[0]
  ident = jnp.eye(_D, dtype=jnp.float32)
  o_ref[0, pl.ds(0, _D), :] = lax.dot_general(
      ident, emb, (((1,), (1,)), ((), ())),
      preferred_element_type=jnp.float32)
  o_ref[0, pl.ds(_D, _NCR), :] = n_ref[0] * s_ref[...] + b_ref[...]


def _card_tc(card_embed_t, nums_t, scale, bias):
  return pl.pallas_call(
      _card_tc_body,
      grid=(_NCARD,),
      in_specs=[
          pl.BlockSpec((1, _B, _D), lambda c: (c, 0, 0)),
          pl.BlockSpec((1, _NCR, _B), lambda c: (c, 0, 0)),
          pl.BlockSpec((_NCR, 1), lambda c: (0, 0)),
          pl.BlockSpec((_NCR, 1), lambda c: (0, 0)),
      ],
      out_specs=pl.BlockSpec((1, _D + _NCR, _B), lambda c: (c, 0, 0)),
      out_shape=jax.ShapeDtypeStruct((_NCARD, _D + _NCR, _B), jnp.float32),
  )(card_embed_t, nums_t, scale, bias)


def _reals_body(r_ref, a_ref, v_ref, o_ref):
  o_ref[...] = (r_ref[...] - a_ref[...]) / jnp.sqrt(v_ref[...])


def _reals_norm(reals, avg, var):
  return pl.pallas_call(
      _reals_body,
      out_shape=jax.ShapeDtypeStruct(reals.shape, reals.dtype),
  )(reals, avg, var)


def kernel(reals, cardIDs, card_nums, actionIDs, action_mask,
           embed_table, avg_reals, var_reals, avg_cards, var_cards):
  cards2 = cardIDs.astype(jnp.int32).T.reshape(_R // 128, 128)
  acts2 = actionIDs.astype(jnp.int32).transpose(1, 2, 0).reshape(_R // 128, 128)
  nums_t = card_nums.transpose(1, 2, 0)
  scale = (1.0 / jnp.sqrt(var_cards)).reshape(_NCR, 1)
  bias = (-avg_cards).reshape(_NCR, 1) * scale
  card_embed_t, act_out = _sc_call(cards2, acts2, embed_table)
  card_out = _card_tc(card_embed_t, nums_t, scale, bias)
  reals_n = _reals_norm(reals, avg_reals, var_reals)
  card_all = card_out.transpose(2, 0, 1)
  action_embed = act_out.transpose(1, 0, 2)
  return (reals_n, card_all, action_embed, action_mask)
